# Initial kernel scaffold; baseline (speedup 1.0000x reference)
import jax
import jax.numpy as jnp
from jax.experimental import pallas as pl


def kernel(xyz_sampled, grid):
    raise NotImplementedError("write your pallas kernel here")


# speedup vs baseline: 2.1445x; 2.1445x over previous
import functools

import jax
import jax.numpy as jnp
from jax import lax
from jax.experimental import pallas as pl
from jax.experimental.pallas import tpu as pltpu
from jax.experimental.pallas import tpu_sc as plsc

RES = 256
N_PTS = 2097152
LANES = 16
BLK = 2048
NC = 2
NS = 16
NW = NC * NS
PW = N_PTS // NW
N_BLOCKS = PW // BLK
SUBV = BLK // LANES


def _tec_body(x_hbm, y_hbm, z_hbm, grid_hbm, out_hbm, *s):
    (xv, yv, zv, fxv, fyv, fzv,
     i000, i001, i010, i011, i100, i101, i110, i111,
     v000, v001, v010, v011, v100, v101, v110, v111,
     outv, sem) = s
    idx_refs = (i000, i001, i010, i011, i100, i101, i110, i111)
    val_refs = (v000, v001, v010, v011, v100, v101, v110, v111)

    wid = lax.axis_index("s") * NC + lax.axis_index("c")

    def block_body(b, carry):
        base = wid * PW + b * BLK
        pltpu.sync_copy(x_hbm.at[pl.ds(base, BLK)], xv)
        pltpu.sync_copy(y_hbm.at[pl.ds(base, BLK)], yv)
        pltpu.sync_copy(z_hbm.at[pl.ds(base, BLK)], zv)

        def pass1(j, c1):
            c = j * LANES
            x = xv[pl.ds(c, LANES)]
            y = yv[pl.ds(c, LANES)]
            z = zv[pl.ds(c, LANES)]
            fscale = jnp.float32(RES - 1)
            ix = (x + 1.0) * 0.5 * fscale
            iy = (y + 1.0) * 0.5 * fscale
            iz = (z + 1.0) * 0.5 * fscale
            xi = ix.astype(jnp.int32)
            yi = iy.astype(jnp.int32)
            zi = iz.astype(jnp.int32)
            fxv[pl.ds(c, LANES)] = ix - xi.astype(jnp.float32)
            fyv[pl.ds(c, LANES)] = iy - yi.astype(jnp.float32)
            fzv[pl.ds(c, LANES)] = iz - zi.astype(jnp.float32)
            fidx = (zi << 16) + (yi << 8) + xi
            i000[pl.ds(c, LANES)] = fidx
            i001[pl.ds(c, LANES)] = fidx + 1
            i010[pl.ds(c, LANES)] = fidx + RES
            i011[pl.ds(c, LANES)] = fidx + (RES + 1)
            i100[pl.ds(c, LANES)] = fidx + RES * RES
            i101[pl.ds(c, LANES)] = fidx + (RES * RES + 1)
            i110[pl.ds(c, LANES)] = fidx + (RES * RES + RES)
            i111[pl.ds(c, LANES)] = fidx + (RES * RES + RES + 1)
            return c1

        lax.fori_loop(0, SUBV, pass1, 0)

        cps = [pltpu.async_copy(grid_hbm.at[idx], val, sem)
               for idx, val in zip(idx_refs, val_refs)]
        for cp in cps:
            cp.wait()

        def pass2(j, c2):
            c = j * LANES
            fx = fxv[pl.ds(c, LANES)]
            fy = fyv[pl.ds(c, LANES)]
            fz = fzv[pl.ds(c, LANES)]
            a000 = v000[pl.ds(c, LANES)]
            a001 = v001[pl.ds(c, LANES)]
            a010 = v010[pl.ds(c, LANES)]
            a011 = v011[pl.ds(c, LANES)]
            a100 = v100[pl.ds(c, LANES)]
            a101 = v101[pl.ds(c, LANES)]
            a110 = v110[pl.ds(c, LANES)]
            a111 = v111[pl.ds(c, LANES)]
            b00 = a000 + fx * (a001 - a000)
            b01 = a010 + fx * (a011 - a010)
            b10 = a100 + fx * (a101 - a100)
            b11 = a110 + fx * (a111 - a110)
            c0 = b00 + fy * (b01 - b00)
            c1 = b10 + fy * (b11 - b10)
            outv[pl.ds(c, LANES)] = c0 + fz * (c1 - c0)
            return c2

        lax.fori_loop(0, SUBV, pass2, 0)

        pltpu.sync_copy(outv, out_hbm.at[pl.ds(base, BLK)])
        return carry

    lax.fori_loop(0, N_BLOCKS, block_body, 0)


def _make_sc_call():
    mesh = plsc.VectorSubcoreMesh(core_axis_name="c", subcore_axis_name="s")
    tile_f = pltpu.VMEM((BLK,), jnp.float32)
    tile_i = pltpu.VMEM((BLK,), jnp.int32)
    scratch = ([tile_f] * 6 + [tile_i] * 8 + [tile_f] * 8 + [tile_f]
               + [pltpu.SemaphoreType.DMA])
    return pl.kernel(
        _tec_body,
        out_type=jax.ShapeDtypeStruct((N_PTS,), jnp.float32),
        mesh=mesh,
        scratch_types=scratch,
    )


_sc_call = _make_sc_call()


@jax.jit
def kernel(xyz_sampled, grid):
    pts = xyz_sampled.reshape(N_PTS, 3)
    gflat = grid.reshape(RES * RES * RES)
    out = _sc_call(pts[:, 0], pts[:, 1], pts[:, 2], gflat)
    return out.reshape(N_PTS)
